# baseline (speedup 1.0000x reference)
import functools

import jax
import jax.numpy as jnp
from jax import lax
from jax.experimental import pallas as pl
from jax.experimental.pallas import tpu as pltpu
from jax.experimental.pallas import tpu_sc as plsc

B = 4096
HIST = 50
D64 = 64
D32 = 32

NC = 2
NS = 16
LANES = 16
NW = NC * NS
ROWS_W = B // NW
NBUF = 5


_CB = 128


def _pack_block(x, d, rows):
    g = 128 // d
    cols = x.shape[1]
    vs = lax.broadcasted_iota(jnp.int32, (rows, cols), 1)
    ps = lax.broadcasted_iota(jnp.int32, (rows, cols), 0)
    outs = []
    for k in range(g):
        e = (vs == g * ps + k).astype(jnp.float32)
        outs.append(lax.dot_general(e, x, (((1,), (1,)), ((), ())),
                                    precision=lax.Precision.HIGHEST,
                                    preferred_element_type=jnp.float32))
    return outs


def _make_repack_body(d, n_full, tail):
    g = 128 // d
    rb = _CB // g

    def body(main_ref, tail_ref, out_ref):
        c = pl.program_id(0)

        @pl.when(c < n_full)
        def _():
            outs = _pack_block(main_ref[...], d, rb)
            for k in range(g):
                out_ref[:, k * d:(k + 1) * d] = outs[k]

        if tail:
            tr = tail // g

            @pl.when(c == n_full)
            def _():
                outs = _pack_block(tail_ref[...], d, tr)
                for k in range(g):
                    out_ref[0:tr, k * d:(k + 1) * d] = outs[k]

    return body


def _repack(w, d, v):
    n_full = v // _CB
    tail = v % _CB
    n_tot = n_full + (1 if tail else 0)
    ob = _CB * d // 128
    wt = w.T
    tail_arr = wt[:, n_full * _CB:] if tail else wt[:, :1]
    out = pl.pallas_call(
        _make_repack_body(d, n_full, tail),
        grid=(n_tot,),
        in_specs=[
            pl.BlockSpec((d, _CB),
                         lambda c: (0, jnp.minimum(c, n_full - 1))),
            pl.BlockSpec(tail_arr.shape, lambda c: (0, 0)),
        ],
        out_specs=pl.BlockSpec((ob, 128), lambda c: (c, 0)),
        out_shape=jax.ShapeDtypeStruct((n_tot * ob, 128), jnp.float32),
    )(wt, tail_arr)
    return out.reshape(n_tot * _CB, d)


def _sc_rows_body(uid, iid, cat, w_user, w_item, w_cat,
                  e_user, e_item, e_cat,
                  vidx, sidx, rbuf, cbuf, rsem):
    c = lax.axis_index("c")
    s = lax.axis_index("s")
    wid = s * NC + c
    base = wid * ROWS_W

    def one(ids, table, buf, out):
        pltpu.sync_copy(ids.at[pl.ds(base, ROWS_W)], vidx)

        @pl.loop(0, ROWS_W // LANES)
        def _(g):
            v = vidx[pl.ds(g * LANES, LANES)]
            for lane in range(LANES):
                r = v[lane]
                pltpu.make_async_copy(
                    table.at[pl.ds(r, 1)],
                    buf.at[pl.ds(g * LANES + lane, 1)], rsem).start()

        pltpu.make_async_copy(table.at[pl.ds(0, ROWS_W)], buf, rsem).wait()
        pltpu.sync_copy(buf, out.at[pl.ds(base, ROWS_W)])

    one(uid, w_user, rbuf, e_user)
    one(iid, w_item, rbuf, e_item)
    one(cat, w_cat, cbuf, e_cat)


def _sc_rows(uid, iid, cat, w_user, w_item, w_cat):
    f32 = jnp.float32
    mesh = plsc.VectorSubcoreMesh(core_axis_name="c", subcore_axis_name="s",
                                  num_cores=NC, num_subcores=NS)
    return pl.kernel(
        _sc_rows_body,
        out_type=(
            jax.ShapeDtypeStruct((B, D64), f32),
            jax.ShapeDtypeStruct((B, D64), f32),
            jax.ShapeDtypeStruct((B, D32), f32),
        ),
        mesh=mesh,
        compiler_params=pltpu.CompilerParams(use_tc_tiling_on_sc=False),
        scratch_types=[
            pltpu.VMEM((ROWS_W,), jnp.int32),
            pltpu.SMEM((ROWS_W,), jnp.int32),
            pltpu.VMEM((ROWS_W, D64), f32),
            pltpu.VMEM((ROWS_W, D32), f32),
            pltpu.SemaphoreType.DMA,
        ],
    )(uid, iid, cat, w_user, w_item, w_cat)


def _sc_gather_body(hist_t, w_hist, hist_sum,
                    hidx, hbufs, myidx, acc, gsems):
    c = lax.axis_index("c")
    s = lax.axis_index("s")
    wid = s * NC + c
    base = wid * ROWS_W

    pltpu.sync_copy(hist_t.at[:, pl.ds(base, ROWS_W)], hidx)

    for k in range(ROWS_W // LANES):
        myidx[pl.ds(k * LANES, LANES)] = (
            lax.iota(jnp.int32, LANES) + (s * ROWS_W + k * LANES))

    def gcopy(jj, bb):
        return pltpu.make_async_copy(w_hist.at[hidx.at[jj]], hbufs[bb],
                                     gsems[bb])

    for bb in range(NBUF):
        gcopy(bb, bb).start()

    @pl.loop(0, HIST, step=NBUF)
    def _ring(j0):
        for bb in range(NBUF):
            j = j0 + bb
            gcopy(j, bb).wait()
            if bb == 0:
                @pl.when(j0 == 0)
                def _():
                    pltpu.sync_copy(hbufs[0], acc.at[myidx])

                @pl.when(j0 > 0)
                def _():
                    pltpu.sync_copy(hbufs[0], acc.at[myidx], add=True)
            else:
                pltpu.sync_copy(hbufs[bb], acc.at[myidx], add=True)

            @pl.when(j + NBUF < HIST)
            def _():
                gcopy(j + NBUF, bb).start()

    pltpu.sync_copy(acc.at[pl.ds(s * ROWS_W, ROWS_W)],
                    hist_sum.at[pl.ds(base, ROWS_W)])


def _sc_gather(hist_t, w_hist):
    f32 = jnp.float32
    mesh = plsc.VectorSubcoreMesh(core_axis_name="c", subcore_axis_name="s",
                                  num_cores=NC, num_subcores=NS)
    return pl.kernel(
        _sc_gather_body,
        out_type=jax.ShapeDtypeStruct((B, D64), f32),
        mesh=mesh,
        compiler_params=pltpu.CompilerParams(use_tc_tiling_on_sc=False),
        scratch_types=[
            pltpu.VMEM((HIST, ROWS_W), jnp.int32),
            [pltpu.VMEM((ROWS_W, D64), f32)] * NBUF,
            pltpu.VMEM((ROWS_W,), jnp.int32),
            pltpu.VMEM_SHARED((NS * ROWS_W, D64), f32),
            [pltpu.SemaphoreType.DMA] * NBUF,
        ],
    )(hist_t, w_hist)


_BLK = 512


def _tc_body(nf, wnum, bnum, eu_r, ei_r, hs_r, ec_r, ve_r, vl_r):
    eu = eu_r[...]
    ei = ei_r[...]
    eh = hs_r[...] * (1.0 / HIST)
    ec = ec_r[...]
    en = nf[...] * wnum[...] + bnum[...]

    ve_r[...] = jnp.concatenate([eu, ei, eh, ec, en], axis=-1)

    def ln(e):
        mu = jnp.mean(e, axis=-1, keepdims=True)
        var = jnp.mean((e - mu) ** 2, axis=-1, keepdims=True)
        return (e - mu) * lax.rsqrt(var + 1e-5)

    vl_r[...] = jnp.concatenate([ln(eu), ln(ei), ln(eh), ln(ec), ln(en)],
                                axis=-1)


def _tc_assemble(num_feat, w_num, b_num, e_user, e_item, hist_sum, e_cat):
    f32 = jnp.float32
    n = B // _BLK
    big = pl.BlockSpec((_BLK, D64), lambda i: (i, 0))
    return pl.pallas_call(
        _tc_body,
        grid=(n,),
        in_specs=[
            pl.BlockSpec((_BLK, 1), lambda i: (i, 0)),
            pl.BlockSpec((1, D32), lambda i: (0, 0)),
            pl.BlockSpec((1, D32), lambda i: (0, 0)),
            big, big, big,
            pl.BlockSpec((_BLK, D32), lambda i: (i, 0)),
        ],
        out_specs=[
            pl.BlockSpec((_BLK, 256), lambda i: (i, 0)),
            pl.BlockSpec((_BLK, 256), lambda i: (i, 0)),
        ],
        out_shape=[
            jax.ShapeDtypeStruct((B, 256), f32),
            jax.ShapeDtypeStruct((B, 256), f32),
        ],
    )(num_feat, w_num, b_num, e_user, e_item, hist_sum, e_cat)


def kernel(user_id, item_id, item_hist, category, num_feat,
           W_user, W_item, W_hist, W_cat, W_num, b_num):
    uid = user_id.astype(jnp.int32)
    iid = item_id.astype(jnp.int32)
    cat = category.astype(jnp.int32)
    hist_t = jnp.transpose(item_hist.astype(jnp.int32))

    v_big = W_hist.shape[0]
    v_cat = W_cat.shape[0]

    wh = _repack(W_hist, D64, v_big)
    hist_sum = _sc_gather(hist_t, wh)

    wu = _repack(W_user, D64, v_big)
    wi = _repack(W_item, D64, v_big)
    wc = _repack(W_cat, D32, v_cat)
    e_user, e_item, e_cat = _sc_rows(uid, iid, cat, wu, wi, wc)

    v_embed, v_embed_ln = _tc_assemble(
        num_feat, W_num, b_num.reshape(1, D32), e_user, e_item, hist_sum,
        e_cat)
    return (v_embed, v_embed_ln)


# speedup vs baseline: 7.8686x; 7.8686x over previous
import functools

import jax
import jax.numpy as jnp
from jax import lax
from jax.experimental import pallas as pl
from jax.experimental.pallas import tpu as pltpu
from jax.experimental.pallas import tpu_sc as plsc

B = 4096
HIST = 50
D64 = 64
D32 = 32

NC = 2
NS = 16
LANES = 16
NW = NC * NS
ROWS_W = B // NW
NBUF = 5


def _sc_rows_body(uid, iid, cat, w_user, w_item, w_cat,
                  e_user, e_item, e_cat,
                  vidx, rbuf, cbuf, rsem):
    c = lax.axis_index("c")
    s = lax.axis_index("s")
    wid = s * NC + c
    base = wid * ROWS_W

    def one(ids, table, buf, out):
        pltpu.sync_copy(ids.at[pl.ds(base, ROWS_W)], vidx)

        @pl.loop(0, ROWS_W // LANES)
        def _(g):
            v = vidx[pl.ds(g * LANES, LANES)]
            for lane in range(LANES):
                r = v[lane]
                pltpu.make_async_copy(
                    table.at[pl.ds(r, 1)],
                    buf.at[pl.ds(g * LANES + lane, 1)], rsem).start()

        pltpu.make_async_copy(table.at[pl.ds(0, ROWS_W)], buf, rsem).wait()
        pltpu.sync_copy(buf, out.at[pl.ds(base, ROWS_W)])

    one(uid, w_user, rbuf, e_user)
    one(iid, w_item, rbuf, e_item)
    one(cat, w_cat, cbuf, e_cat)


def _sc_rows(uid, iid, cat, w_user, w_item, w_cat):
    f32 = jnp.float32
    mesh = plsc.VectorSubcoreMesh(core_axis_name="c", subcore_axis_name="s",
                                  num_cores=NC, num_subcores=NS)
    return pl.kernel(
        _sc_rows_body,
        out_type=(
            jax.ShapeDtypeStruct((B, D64), f32),
            jax.ShapeDtypeStruct((B, D64), f32),
            jax.ShapeDtypeStruct((B, D32), f32),
        ),
        mesh=mesh,
        compiler_params=pltpu.CompilerParams(use_tc_tiling_on_sc=True),
        scratch_types=[
            pltpu.VMEM((ROWS_W,), jnp.int32),
            pltpu.VMEM((ROWS_W, D64), f32),
            pltpu.VMEM((ROWS_W, D32), f32),
            pltpu.SemaphoreType.DMA,
        ],
    )(uid, iid, cat, w_user, w_item, w_cat)


def _sc_gather_body(hist_t, w_hist, hist_sum,
                    hidx, hbufs, myidx, acc, gsems):
    c = lax.axis_index("c")
    s = lax.axis_index("s")
    wid = s * NC + c
    base = wid * ROWS_W

    pltpu.sync_copy(hist_t.at[:, pl.ds(base, ROWS_W)], hidx)

    for k in range(ROWS_W // LANES):
        myidx[pl.ds(k * LANES, LANES)] = (
            lax.iota(jnp.int32, LANES) + (s * ROWS_W + k * LANES))

    def gcopy(jj, bb):
        return pltpu.make_async_copy(w_hist.at[hidx.at[jj]], hbufs[bb],
                                     gsems[bb])

    for bb in range(NBUF):
        gcopy(bb, bb).start()

    @pl.loop(0, HIST, step=NBUF)
    def _ring(j0):
        for bb in range(NBUF):
            j = j0 + bb
            gcopy(j, bb).wait()
            if bb == 0:
                @pl.when(j0 == 0)
                def _():
                    pltpu.sync_copy(hbufs[0], acc.at[myidx])

                @pl.when(j0 > 0)
                def _():
                    pltpu.sync_copy(hbufs[0], acc.at[myidx], add=True)
            else:
                pltpu.sync_copy(hbufs[bb], acc.at[myidx], add=True)

            @pl.when(j + NBUF < HIST)
            def _():
                gcopy(j + NBUF, bb).start()

    pltpu.sync_copy(acc.at[pl.ds(s * ROWS_W, ROWS_W)],
                    hist_sum.at[pl.ds(base, ROWS_W)])


def _sc_gather(hist_t, w_hist):
    f32 = jnp.float32
    mesh = plsc.VectorSubcoreMesh(core_axis_name="c", subcore_axis_name="s",
                                  num_cores=NC, num_subcores=NS)
    return pl.kernel(
        _sc_gather_body,
        out_type=jax.ShapeDtypeStruct((B, D64), f32),
        mesh=mesh,
        compiler_params=pltpu.CompilerParams(use_tc_tiling_on_sc=False),
        scratch_types=[
            pltpu.VMEM((HIST, ROWS_W), jnp.int32),
            [pltpu.VMEM((ROWS_W, D64), f32)] * NBUF,
            pltpu.VMEM((ROWS_W,), jnp.int32),
            pltpu.VMEM_SHARED((NS * ROWS_W, D64), f32),
            [pltpu.SemaphoreType.DMA] * NBUF,
        ],
    )(hist_t, w_hist)


_BLK = 512


def _tc_body(nf, wnum, bnum, eu_r, ei_r, hs_r, ec_r, ve_r, vl_r):
    eu = eu_r[...]
    ei = ei_r[...]
    eh = hs_r[...] * (1.0 / HIST)
    ec = ec_r[...]
    en = nf[...] * wnum[...] + bnum[...]

    ve_r[...] = jnp.concatenate([eu, ei, eh, ec, en], axis=-1)

    def ln(e):
        mu = jnp.mean(e, axis=-1, keepdims=True)
        var = jnp.mean((e - mu) ** 2, axis=-1, keepdims=True)
        return (e - mu) * lax.rsqrt(var + 1e-5)

    vl_r[...] = jnp.concatenate([ln(eu), ln(ei), ln(eh), ln(ec), ln(en)],
                                axis=-1)


def _tc_assemble(num_feat, w_num, b_num, e_user, e_item, hist_sum, e_cat):
    f32 = jnp.float32
    n = B // _BLK
    big = pl.BlockSpec((_BLK, D64), lambda i: (i, 0))
    return pl.pallas_call(
        _tc_body,
        grid=(n,),
        in_specs=[
            pl.BlockSpec((_BLK, 1), lambda i: (i, 0)),
            pl.BlockSpec((1, D32), lambda i: (0, 0)),
            pl.BlockSpec((1, D32), lambda i: (0, 0)),
            big, big, big,
            pl.BlockSpec((_BLK, D32), lambda i: (i, 0)),
        ],
        out_specs=[
            pl.BlockSpec((_BLK, 256), lambda i: (i, 0)),
            pl.BlockSpec((_BLK, 256), lambda i: (i, 0)),
        ],
        out_shape=[
            jax.ShapeDtypeStruct((B, 256), f32),
            jax.ShapeDtypeStruct((B, 256), f32),
        ],
    )(num_feat, w_num, b_num, e_user, e_item, hist_sum, e_cat)


def kernel(user_id, item_id, item_hist, category, num_feat,
           W_user, W_item, W_hist, W_cat, W_num, b_num):
    uid = user_id.astype(jnp.int32)
    iid = item_id.astype(jnp.int32)
    cat = category.astype(jnp.int32)
    hist_t = jnp.transpose(item_hist.astype(jnp.int32))

    hist_sum = _sc_gather(hist_t, W_hist)
    e_user, e_item, e_cat = _sc_rows(uid, iid, cat, W_user, W_item, W_cat)

    v_embed, v_embed_ln = _tc_assemble(
        num_feat, W_num, b_num.reshape(1, D32), e_user, e_item, hist_sum,
        e_cat)
    return (v_embed, v_embed_ln)


# speedup vs baseline: 7.8860x; 1.0022x over previous
import functools

import jax
import jax.numpy as jnp
from jax import lax
from jax.experimental import pallas as pl
from jax.experimental.pallas import tpu as pltpu
from jax.experimental.pallas import tpu_sc as plsc

B = 4096
HIST = 50
D64 = 64
D32 = 32

NC = 2
NS = 16
LANES = 16
NW = NC * NS
ROWS_W = B // NW
NBUF = 10


def _sc_rows_body(uid, iid, cat, w_user, w_item, w_cat,
                  e_user, e_item, e_cat,
                  vidx, rbuf, cbuf, rsem):
    c = lax.axis_index("c")
    s = lax.axis_index("s")
    wid = s * NC + c
    base = wid * ROWS_W

    def one(ids, table, buf, out):
        pltpu.sync_copy(ids.at[pl.ds(base, ROWS_W)], vidx)

        @pl.loop(0, ROWS_W // LANES)
        def _(g):
            v = vidx[pl.ds(g * LANES, LANES)]
            for lane in range(LANES):
                r = v[lane]
                pltpu.make_async_copy(
                    table.at[pl.ds(r, 1)],
                    buf.at[pl.ds(g * LANES + lane, 1)], rsem).start()

        pltpu.make_async_copy(table.at[pl.ds(0, ROWS_W)], buf, rsem).wait()
        pltpu.sync_copy(buf, out.at[pl.ds(base, ROWS_W)])

    one(uid, w_user, rbuf, e_user)
    one(iid, w_item, rbuf, e_item)
    one(cat, w_cat, cbuf, e_cat)


def _sc_rows(uid, iid, cat, w_user, w_item, w_cat):
    f32 = jnp.float32
    mesh = plsc.VectorSubcoreMesh(core_axis_name="c", subcore_axis_name="s",
                                  num_cores=NC, num_subcores=NS)
    return pl.kernel(
        _sc_rows_body,
        out_type=(
            jax.ShapeDtypeStruct((B, D64), f32),
            jax.ShapeDtypeStruct((B, D64), f32),
            jax.ShapeDtypeStruct((B, D32), f32),
        ),
        mesh=mesh,
        compiler_params=pltpu.CompilerParams(use_tc_tiling_on_sc=True),
        scratch_types=[
            pltpu.VMEM((ROWS_W,), jnp.int32),
            pltpu.VMEM((ROWS_W, D64), f32),
            pltpu.VMEM((ROWS_W, D32), f32),
            pltpu.SemaphoreType.DMA,
        ],
    )(uid, iid, cat, w_user, w_item, w_cat)


def _sc_gather_body(hist_t, w_hist, hist_sum,
                    hidx, hbufs, myidx, acc, gsems):
    c = lax.axis_index("c")
    s = lax.axis_index("s")
    wid = s * NC + c
    base = wid * ROWS_W

    pltpu.sync_copy(hist_t.at[:, pl.ds(base, ROWS_W)], hidx)

    for k in range(ROWS_W // LANES):
        myidx[pl.ds(k * LANES, LANES)] = (
            lax.iota(jnp.int32, LANES) + (s * ROWS_W + k * LANES))

    def gcopy(jj, bb):
        return pltpu.make_async_copy(w_hist.at[hidx.at[jj]], hbufs[bb],
                                     gsems[bb])

    for bb in range(NBUF):
        gcopy(bb, bb).start()

    @pl.loop(0, HIST, step=NBUF)
    def _ring(j0):
        for bb in range(NBUF):
            j = j0 + bb
            gcopy(j, bb).wait()
            if bb == 0:
                @pl.when(j0 == 0)
                def _():
                    pltpu.sync_copy(hbufs[0], acc.at[myidx])

                @pl.when(j0 > 0)
                def _():
                    pltpu.sync_copy(hbufs[0], acc.at[myidx], add=True)
            else:
                pltpu.sync_copy(hbufs[bb], acc.at[myidx], add=True)

            @pl.when(j + NBUF < HIST)
            def _():
                gcopy(j + NBUF, bb).start()

    pltpu.sync_copy(acc.at[pl.ds(s * ROWS_W, ROWS_W)],
                    hist_sum.at[pl.ds(base, ROWS_W)])


def _sc_gather(hist_t, w_hist):
    f32 = jnp.float32
    mesh = plsc.VectorSubcoreMesh(core_axis_name="c", subcore_axis_name="s",
                                  num_cores=NC, num_subcores=NS)
    return pl.kernel(
        _sc_gather_body,
        out_type=jax.ShapeDtypeStruct((B, D64), f32),
        mesh=mesh,
        compiler_params=pltpu.CompilerParams(use_tc_tiling_on_sc=False),
        scratch_types=[
            pltpu.VMEM((HIST, ROWS_W), jnp.int32),
            [pltpu.VMEM((ROWS_W, D64), f32)] * NBUF,
            pltpu.VMEM((ROWS_W,), jnp.int32),
            pltpu.VMEM_SHARED((NS * ROWS_W, D64), f32),
            [pltpu.SemaphoreType.DMA] * NBUF,
        ],
    )(hist_t, w_hist)


_BLK = 512


def _tc_body(nf, wnum, bnum, eu_r, ei_r, hs_r, ec_r, ve_r, vl_r):
    eu = eu_r[...]
    ei = ei_r[...]
    eh = hs_r[...] * (1.0 / HIST)
    ec = ec_r[...]
    en = nf[...] * wnum[...] + bnum[...]

    ve_r[...] = jnp.concatenate([eu, ei, eh, ec, en], axis=-1)

    def ln(e):
        mu = jnp.mean(e, axis=-1, keepdims=True)
        var = jnp.mean((e - mu) ** 2, axis=-1, keepdims=True)
        return (e - mu) * lax.rsqrt(var + 1e-5)

    vl_r[...] = jnp.concatenate([ln(eu), ln(ei), ln(eh), ln(ec), ln(en)],
                                axis=-1)


def _tc_assemble(num_feat, w_num, b_num, e_user, e_item, hist_sum, e_cat):
    f32 = jnp.float32
    n = B // _BLK
    big = pl.BlockSpec((_BLK, D64), lambda i: (i, 0))
    return pl.pallas_call(
        _tc_body,
        grid=(n,),
        in_specs=[
            pl.BlockSpec((_BLK, 1), lambda i: (i, 0)),
            pl.BlockSpec((1, D32), lambda i: (0, 0)),
            pl.BlockSpec((1, D32), lambda i: (0, 0)),
            big, big, big,
            pl.BlockSpec((_BLK, D32), lambda i: (i, 0)),
        ],
        out_specs=[
            pl.BlockSpec((_BLK, 256), lambda i: (i, 0)),
            pl.BlockSpec((_BLK, 256), lambda i: (i, 0)),
        ],
        out_shape=[
            jax.ShapeDtypeStruct((B, 256), f32),
            jax.ShapeDtypeStruct((B, 256), f32),
        ],
    )(num_feat, w_num, b_num, e_user, e_item, hist_sum, e_cat)


def kernel(user_id, item_id, item_hist, category, num_feat,
           W_user, W_item, W_hist, W_cat, W_num, b_num):
    uid = user_id.astype(jnp.int32)
    iid = item_id.astype(jnp.int32)
    cat = category.astype(jnp.int32)
    hist_t = jnp.transpose(item_hist.astype(jnp.int32))

    hist_sum = _sc_gather(hist_t, W_hist)
    e_user, e_item, e_cat = _sc_rows(uid, iid, cat, W_user, W_item, W_cat)

    v_embed, v_embed_ln = _tc_assemble(
        num_feat, W_num, b_num.reshape(1, D32), e_user, e_item, hist_sum,
        e_cat)
    return (v_embed, v_embed_ln)
